# baseline (device time: 429410 ns/iter reference)
import jax
import jax.numpy as jnp
from jax import lax
from jax.experimental import pallas as pl
from jax.experimental.pallas import tpu as pltpu

M_CHUNK = 256
K_CHUNK = 512


def kernel(A, B):
    m, k = A.shape
    k2, n = B.shape
    assert k == k2
    num_m = m // M_CHUNK
    num_k = k // K_CHUNK

    def body(A_ref, B_ref, out_ref, send_buf, wire_buf, comm_buf,
             send_sems, recv_sems):
        i = pl.program_id(0)
        j = pl.program_id(1)
        my_x = lax.axis_index("x")
        my_y = lax.axis_index("y")
        nbr = (my_x, 1 - my_y)

        @pl.when((i == 0) & (j == 0))
        def _():
            barrier = pltpu.get_barrier_semaphore()
            pl.semaphore_signal(barrier, inc=1, device_id=nbr,
                                device_id_type=pl.DeviceIdType.MESH)
            pl.semaphore_wait(barrier, 1)

        acc = lax.rem(i, 2)

        @pl.when((j == 0) & (i > 1))
        def _():
            prev = i - 2
            pslot = lax.rem(prev, 4)
            prev_rdma = pltpu.make_async_remote_copy(
                src_ref=wire_buf.at[acc],
                dst_ref=comm_buf.at[pslot],
                send_sem=send_sems.at[pslot],
                recv_sem=recv_sems.at[pslot],
                device_id=nbr,
                device_id_type=pl.DeviceIdType.MESH,
            )
            prev_rdma.wait()
            out_ref[...] = send_buf[acc] + comm_buf[pslot].astype(jnp.float32)

        @pl.when(i < num_m)
        def _():
            partial = jnp.dot(A_ref[...], B_ref[...],
                              preferred_element_type=jnp.float32)

            @pl.when(j == 0)
            def _():
                send_buf[acc] = partial

            @pl.when(j > 0)
            def _():
                send_buf[acc] += partial

        @pl.when((j == num_k - 1) & (i < num_m))
        def _():
            slot = lax.rem(i, 4)
            wire_buf[acc] = send_buf[acc].astype(jnp.bfloat16)
            rdma = pltpu.make_async_remote_copy(
                src_ref=wire_buf.at[acc],
                dst_ref=comm_buf.at[slot],
                send_sem=send_sems.at[slot],
                recv_sem=recv_sems.at[slot],
                device_id=nbr,
                device_id_type=pl.DeviceIdType.MESH,
            )
            rdma.start()

    grid = (num_m + 2, num_k)
    last_m = num_m - 1
    return pl.pallas_call(
        body,
        grid=grid,
        out_shape=jax.ShapeDtypeStruct((m, n), jnp.float32),
        in_specs=[
            pl.BlockSpec((M_CHUNK, K_CHUNK),
                         lambda i, j: (jnp.minimum(i, last_m), j)),
            pl.BlockSpec((K_CHUNK, n), lambda i, j: (j, 0)),
        ],
        out_specs=pl.BlockSpec((M_CHUNK, n),
                               lambda i, j: (jnp.maximum(i - 2, 0), 0)),
        scratch_shapes=[
            pltpu.VMEM((2, M_CHUNK, n), jnp.float32),
            pltpu.VMEM((2, M_CHUNK, n), jnp.bfloat16),
            pltpu.VMEM((4, M_CHUNK, n), jnp.bfloat16),
            pltpu.SemaphoreType.DMA((4,)),
            pltpu.SemaphoreType.DMA((4,)),
        ],
        compiler_params=pltpu.CompilerParams(
            collective_id=0,
            dimension_semantics=("arbitrary", "arbitrary"),
            vmem_limit_bytes=60 * 1024 * 1024,
        ),
    )(A, B)


# device time: 427237 ns/iter; 1.0051x vs baseline; 1.0051x over previous
import jax
import jax.numpy as jnp
from jax import lax
from jax.experimental import pallas as pl
from jax.experimental.pallas import tpu as pltpu

M_CHUNK = 256
B_SLAB = 256


def kernel(A, B):
    m, k = A.shape
    k2, n = B.shape
    assert k == k2
    num_m = m // M_CHUNK
    num_slabs = k // B_SLAB

    def body(A_ref, B_hbm, out_ref, b16, btmp, send_buf, wire_buf, comm_buf,
             init_sems, send_sems, recv_sems):
        i = pl.program_id(0)
        my_x = lax.axis_index("x")
        my_y = lax.axis_index("y")
        nbr = (my_x, 1 - my_y)

        @pl.when(i == 0)
        def _():
            barrier = pltpu.get_barrier_semaphore()
            pl.semaphore_signal(barrier, inc=1, device_id=nbr,
                                device_id_type=pl.DeviceIdType.MESH)
            pl.semaphore_wait(barrier, 1)

            def slab_copy(t):
                return pltpu.make_async_copy(
                    B_hbm.at[pl.ds(t * B_SLAB, B_SLAB), :],
                    btmp.at[t % 2],
                    init_sems.at[t % 2],
                )

            slab_copy(0).start()
            for t in range(1, num_slabs):
                slab_copy(t).start()
                slab_copy(t - 1).wait()
                b16[pl.ds((t - 1) * B_SLAB, B_SLAB), :] = (
                    btmp[(t - 1) % 2].astype(jnp.bfloat16))
            slab_copy(num_slabs - 1).wait()
            b16[pl.ds((num_slabs - 1) * B_SLAB, B_SLAB), :] = (
                btmp[(num_slabs - 1) % 2].astype(jnp.bfloat16))

        acc = lax.rem(i, 2)

        @pl.when(i < num_m)
        def _():
            partial = jnp.dot(A_ref[...].astype(jnp.bfloat16), b16[...],
                              preferred_element_type=jnp.float32)
            send_buf[acc] = partial
            wire_buf[acc] = partial.astype(jnp.bfloat16)
            slot = lax.rem(i, 4)
            rdma = pltpu.make_async_remote_copy(
                src_ref=wire_buf.at[acc],
                dst_ref=comm_buf.at[slot],
                send_sem=send_sems.at[slot],
                recv_sem=recv_sems.at[slot],
                device_id=nbr,
                device_id_type=pl.DeviceIdType.MESH,
            )
            rdma.start()

        @pl.when(i > 0)
        def _():
            prev = i - 1
            pacc = lax.rem(prev, 2)
            pslot = lax.rem(prev, 4)
            prev_rdma = pltpu.make_async_remote_copy(
                src_ref=wire_buf.at[pacc],
                dst_ref=comm_buf.at[pslot],
                send_sem=send_sems.at[pslot],
                recv_sem=recv_sems.at[pslot],
                device_id=nbr,
                device_id_type=pl.DeviceIdType.MESH,
            )
            prev_rdma.wait()
            out_ref[...] = send_buf[pacc] + comm_buf[pslot].astype(jnp.float32)

    grid = (num_m + 1,)
    last_m = num_m - 1
    return pl.pallas_call(
        body,
        grid=grid,
        out_shape=jax.ShapeDtypeStruct((m, n), jnp.float32),
        in_specs=[
            pl.BlockSpec((M_CHUNK, k), lambda i: (jnp.minimum(i, last_m), 0)),
            pl.BlockSpec(memory_space=pltpu.MemorySpace.HBM),
        ],
        out_specs=pl.BlockSpec((M_CHUNK, n),
                               lambda i: (jnp.maximum(i - 1, 0), 0)),
        scratch_shapes=[
            pltpu.VMEM((k, n), jnp.bfloat16),
            pltpu.VMEM((2, B_SLAB, n), jnp.float32),
            pltpu.VMEM((2, M_CHUNK, n), jnp.float32),
            pltpu.VMEM((2, M_CHUNK, n), jnp.bfloat16),
            pltpu.VMEM((4, M_CHUNK, n), jnp.bfloat16),
            pltpu.SemaphoreType.DMA((2,)),
            pltpu.SemaphoreType.DMA((4,)),
            pltpu.SemaphoreType.DMA((4,)),
        ],
        compiler_params=pltpu.CompilerParams(
            collective_id=0,
            dimension_semantics=("arbitrary",),
            vmem_limit_bytes=60 * 1024 * 1024,
        ),
    )(A, B)


# device time: 423173 ns/iter; 1.0147x vs baseline; 1.0096x over previous
import jax
import jax.numpy as jnp
from jax import lax
from jax.experimental import pallas as pl
from jax.experimental.pallas import tpu as pltpu

M_CHUNK = 256
K_CHUNK = 512


def kernel(A, B):
    m, k = A.shape
    k2, n = B.shape
    assert k == k2
    num_m = m // M_CHUNK
    num_k = k // K_CHUNK

    def body(A_ref, B_ref, out_ref, send_buf, wire_buf, comm_buf,
             send_sems, recv_sems):
        i = pl.program_id(0)
        j = pl.program_id(1)
        my_x = lax.axis_index("x")
        my_y = lax.axis_index("y")
        nbr = (my_x, 1 - my_y)

        @pl.when((i == 0) & (j == 0))
        def _():
            barrier = pltpu.get_barrier_semaphore()
            pl.semaphore_signal(barrier, inc=1, device_id=nbr,
                                device_id_type=pl.DeviceIdType.MESH)
            pl.semaphore_wait(barrier, 1)

        acc = lax.rem(i, 2)

        @pl.when(i < num_m)
        def _():
            partial = jnp.dot(A_ref[...], B_ref[...],
                              preferred_element_type=jnp.float32)

            @pl.when(j == 0)
            def _():
                send_buf[acc] = partial

            @pl.when(j > 0)
            def _():
                send_buf[acc] += partial

        @pl.when(j == num_k - 1)
        def _():
            @pl.when(i < num_m)
            def _():
                slot = lax.rem(i, 4)
                wire_buf[acc] = send_buf[acc].astype(jnp.bfloat16)
                rdma = pltpu.make_async_remote_copy(
                    src_ref=wire_buf.at[acc],
                    dst_ref=comm_buf.at[slot],
                    send_sem=send_sems.at[slot],
                    recv_sem=recv_sems.at[slot],
                    device_id=nbr,
                    device_id_type=pl.DeviceIdType.MESH,
                )
                rdma.start()

            @pl.when(i > 0)
            def _():
                prev = i - 1
                pacc = lax.rem(prev, 2)
                pslot = lax.rem(prev, 4)
                prev_rdma = pltpu.make_async_remote_copy(
                    src_ref=wire_buf.at[pacc],
                    dst_ref=comm_buf.at[pslot],
                    send_sem=send_sems.at[pslot],
                    recv_sem=recv_sems.at[pslot],
                    device_id=nbr,
                    device_id_type=pl.DeviceIdType.MESH,
                )
                prev_rdma.wait()
                out_ref[...] = send_buf[pacc] + comm_buf[pslot].astype(
                    jnp.float32)

    grid = (num_m + 1, num_k)
    last_m = num_m - 1
    return pl.pallas_call(
        body,
        grid=grid,
        out_shape=jax.ShapeDtypeStruct((m, n), jnp.float32),
        in_specs=[
            pl.BlockSpec((M_CHUNK, K_CHUNK),
                         lambda i, j: (jnp.minimum(i, last_m), j)),
            pl.BlockSpec((K_CHUNK, n), lambda i, j: (j, 0)),
        ],
        out_specs=pl.BlockSpec((M_CHUNK, n),
                               lambda i, j: (jnp.maximum(i - 1, 0), 0)),
        scratch_shapes=[
            pltpu.VMEM((2, M_CHUNK, n), jnp.float32),
            pltpu.VMEM((2, M_CHUNK, n), jnp.bfloat16),
            pltpu.VMEM((4, M_CHUNK, n), jnp.bfloat16),
            pltpu.SemaphoreType.DMA((4,)),
            pltpu.SemaphoreType.DMA((4,)),
        ],
        compiler_params=pltpu.CompilerParams(
            collective_id=0,
            dimension_semantics=("arbitrary", "arbitrary"),
            vmem_limit_bytes=60 * 1024 * 1024,
        ),
    )(A, B)
